# baseline (speedup 1.0000x reference)
import functools

import jax
import jax.numpy as jnp
from jax import lax
from jax.experimental import pallas as pl
from jax.experimental.pallas import tpu as pltpu
from jax.experimental.pallas import tpu_sc as plsc

_info = plsc.get_sparse_core_info()
_NC, _NS = _info.num_cores, _info.num_subcores
_NW = _NC * _NS

_CHUNK = 400
_NBUF = 2
_NROW = 3


@functools.partial(jax.jit, static_argnums=(2, 3))
def _sc_gather(tok, table, bpw, d):
    s, b = tok.shape
    n_chunks = bpw // _CHUNK
    n_outer = n_chunks // _NBUF
    assert bpw % _CHUNK == 0 and n_chunks % _NBUF == 0
    assert (_NROW - 1) * b >= bpw
    mesh = plsc.VectorSubcoreMesh(core_axis_name="c", subcore_axis_name="s")

    @functools.partial(
        pl.kernel,
        mesh=mesh,
        out_type=jax.ShapeDtypeStruct((s * b, d), jnp.float32),
        scratch_types=[
            pltpu.VMEM((_NROW * b,), jnp.int32),
            pltpu.VMEM((_NBUF, _CHUNK, d), jnp.float32),
            pltpu.SemaphoreType.DMA,
            pltpu.SemaphoreType.DMA,
            pltpu.SemaphoreType.DMA,
        ],
    )
    def k(tok_hbm, table_hbm, out_hbm, idx_v, rows_v, s0, s1, ws):
        gsems = (s0, s1)
        wid = lax.axis_index("s") * _NC + lax.axis_index("c")
        base = wid * bpw
        r0 = base // b
        c0 = base - r0 * b
        stage = []
        for kk in range(_NROW):
            rk = jnp.minimum(r0 + kk, s - 1)
            stage.append(
                pltpu.async_copy(tok_hbm.at[rk], idx_v.at[pl.ds(kk * b, b)], ws)
            )
        for cp in stage:
            cp.wait()

        def body(i, carry):
            ioff = i * (_NBUF * _CHUNK)
            gets = []
            for bb in range(_NBUF):
                off = pl.multiple_of(c0 + ioff + bb * _CHUNK, 8)
                gets.append(
                    pltpu.async_copy(
                        table_hbm.at[idx_v.at[pl.ds(off, _CHUNK)]],
                        rows_v.at[bb],
                        gsems[bb],
                    )
                )
            puts = []
            for bb in range(_NBUF):
                off = pl.multiple_of(ioff + bb * _CHUNK, 8)
                gets[bb].wait()
                puts.append(
                    pltpu.async_copy(
                        rows_v.at[bb], out_hbm.at[pl.ds(base + off, _CHUNK)], ws
                    )
                )
            for p in puts:
                p.wait()
            return carry

        lax.fori_loop(0, n_outer, body, 0)

    return k(tok, table)


def kernel(token_ids, embed_matrix):
    b, s = token_ids.shape
    v, d = embed_matrix.shape
    tok = token_ids.T.astype(jnp.int32)
    bpw = (b * s) // _NW
    out = _sc_gather(tok, embed_matrix, bpw, d)
    return out.reshape(s, b, d).transpose(1, 0, 2)
